# baseline (speedup 1.0000x reference)
import functools

import jax
import jax.numpy as jnp
from jax.experimental import pallas as pl
from jax.experimental.pallas import tpu as pltpu

_DEPTH = 4


def _se_ring_body(x_hbm, w1t_ref, w2t_ref, o_hbm, x_buf, o_buf,
                  in_sems, out_sems, *, bt, steps, inv_hw):
    base = pl.program_id(0) * steps

    def dma_in(slot, step):
        pltpu.make_async_copy(x_hbm.at[pl.ds((base + step) * bt, bt)],
                              x_buf.at[slot], in_sems.at[slot]).start()

    def wait_in(slot):
        pltpu.make_async_copy(x_buf.at[slot], x_buf.at[slot],
                              in_sems.at[slot]).wait()

    def dma_out(slot, step):
        pltpu.make_async_copy(o_buf.at[slot],
                              o_hbm.at[pl.ds((base + step) * bt, bt)],
                              out_sems.at[slot]).start(priority=1)

    def wait_out(slot):
        pltpu.make_async_copy(o_buf.at[slot], o_buf.at[slot],
                              out_sems.at[slot]).wait()

    for k in range(min(_DEPTH, steps)):
        dma_in(k, k)

    for i in range(steps):
        slot = i % _DEPTH
        wait_in(slot)
        if i >= _DEPTH:
            wait_out(slot)

        x = x_buf[slot]
        pooled = jnp.sum(x, axis=2, dtype=jnp.float32) * inv_hw
        hid = jnp.maximum(
            jnp.dot(pooled, w1t_ref[...],
                    preferred_element_type=jnp.float32), 0.0)
        gate = jax.nn.sigmoid(
            jnp.dot(hid, w2t_ref[...], preferred_element_type=jnp.float32))
        o_buf[slot] = x * gate.astype(x.dtype)[:, :, None]

        dma_out(slot, i)
        if i + _DEPTH < steps:
            dma_in(slot, i + _DEPTH)

    for i in range(max(steps - _DEPTH, 0), steps):
        wait_out(i % _DEPTH)


def _se_slab_body(x_ref, w1t_ref, w2t_ref, o_ref, *, inv_hw):
    x = x_ref[...]
    pooled = jnp.sum(x, axis=2, dtype=jnp.float32) * inv_hw
    hid = jnp.maximum(
        jnp.dot(pooled, w1t_ref[...], preferred_element_type=jnp.float32), 0.0)
    gate = jax.nn.sigmoid(
        jnp.dot(hid, w2t_ref[...], preferred_element_type=jnp.float32))
    o_ref[...] = x * gate.astype(x.dtype)[:, :, None]


def kernel(x_nchw, w1, w2):
    B, C, H, W = x_nchw.shape
    hidden = w1.shape[0]
    HW = H * W
    dt = x_nchw.dtype
    itemsize = jnp.dtype(dt).itemsize

    x_flat = x_nchw.reshape(B, C, HW)
    w1t = w1.T.astype(jnp.float32)
    w2t = w2.T.astype(jnp.float32)
    inv_hw = 1.0 / HW

    cost = pl.CostEstimate(
        flops=B * (4 * C * HW + 4 * C * hidden),
        transcendentals=B * C,
        bytes_accessed=2 * B * C * HW * itemsize,
    )

    bt = 2
    nA = (int(round(B * 0.44)) // (2 * bt)) * (2 * bt)
    nA = min(max(nA, 0), B)
    HWp = -(-HW // 128) * 128
    nB = B - nA

    xb = (jnp.pad(x_flat[nA:], ((0, 0), (0, 0), (0, HWp - HW)))
          if nB > 0 else None)

    if nA > 0:
        steps = nA // (bt * 2)
        out_a = pl.pallas_call(
            functools.partial(_se_ring_body, bt=bt, steps=steps,
                              inv_hw=inv_hw),
            out_shape=jax.ShapeDtypeStruct((B, C, HW), dt),
            grid=(2,),
            in_specs=[
                pl.BlockSpec(memory_space=pl.ANY),
                pl.BlockSpec((C, hidden), lambda p: (0, 0)),
                pl.BlockSpec((hidden, C), lambda p: (0, 0)),
            ],
            out_specs=pl.BlockSpec(memory_space=pl.ANY),
            scratch_shapes=[
                pltpu.VMEM((_DEPTH, bt, C, HW), dt),
                pltpu.VMEM((_DEPTH, bt, C, HW), dt),
                pltpu.SemaphoreType.DMA((_DEPTH,)),
                pltpu.SemaphoreType.DMA((_DEPTH,)),
            ],
            compiler_params=pltpu.CompilerParams(
                dimension_semantics=("parallel",),
                vmem_limit_bytes=60 << 20,
            ),
            cost_estimate=cost,
        )(x_flat, w1t, w2t)
    else:
        out_a = None

    if nB > 0:
        btb = 8
        while nB % btb:
            btb -= 1
        out_bp = pl.pallas_call(
            functools.partial(_se_slab_body, inv_hw=inv_hw),
            out_shape=jax.ShapeDtypeStruct((nB, C, HWp), dt),
            grid=(nB // btb,),
            in_specs=[
                pl.BlockSpec((btb, C, HWp), lambda b: (b, 0, 0)),
                pl.BlockSpec((C, hidden), lambda b: (0, 0)),
                pl.BlockSpec((hidden, C), lambda b: (0, 0)),
            ],
            out_specs=pl.BlockSpec((btb, C, HWp), lambda b: (b, 0, 0)),
            compiler_params=pltpu.CompilerParams(
                dimension_semantics=("parallel",),
                vmem_limit_bytes=60 << 20,
            ),
            cost_estimate=cost,
        )(xb, w1t, w2t)
        out_b = out_bp[:, :, :HW]
        if out_a is None:
            out_flat = out_b
        else:
            out_flat = jax.lax.dynamic_update_slice(out_a, out_b, (nA, 0, 0))
    else:
        out_flat = out_a

    return out_flat.reshape(B, C, H, W)


# speedup vs baseline: 1.4784x; 1.4784x over previous
import functools

import jax
import jax.numpy as jnp
from jax.experimental import pallas as pl
from jax.experimental.pallas import tpu as pltpu

_DEPTH = 4


def _se_ring_body(x_hbm, w1t_ref, w2t_ref, o_hbm, x_buf, o_buf,
                  in_sems, out_sems, *, bt, steps, inv_hw):
    base = pl.program_id(0) * steps

    def dma_in(slot, step):
        pltpu.make_async_copy(x_hbm.at[pl.ds((base + step) * bt, bt)],
                              x_buf.at[slot], in_sems.at[slot]).start()

    def wait_in(slot):
        pltpu.make_async_copy(x_buf.at[slot], x_buf.at[slot],
                              in_sems.at[slot]).wait()

    def dma_out(slot, step):
        pltpu.make_async_copy(o_buf.at[slot],
                              o_hbm.at[pl.ds((base + step) * bt, bt)],
                              out_sems.at[slot]).start(priority=1)

    def wait_out(slot):
        pltpu.make_async_copy(o_buf.at[slot], o_buf.at[slot],
                              out_sems.at[slot]).wait()

    for k in range(min(_DEPTH, steps)):
        dma_in(k, k)

    for i in range(steps):
        slot = i % _DEPTH
        wait_in(slot)
        if i >= _DEPTH:
            wait_out(slot)

        x = x_buf[slot]
        pooled = jnp.sum(x, axis=2, dtype=jnp.float32) * inv_hw
        hid = jnp.maximum(
            jnp.dot(pooled, w1t_ref[...],
                    preferred_element_type=jnp.float32), 0.0)
        gate = jax.nn.sigmoid(
            jnp.dot(hid, w2t_ref[...], preferred_element_type=jnp.float32))
        o_buf[slot] = x * gate.astype(x.dtype)[:, :, None]

        dma_out(slot, i)
        if i + _DEPTH < steps:
            dma_in(slot, i + _DEPTH)

    for i in range(max(steps - _DEPTH, 0), steps):
        wait_out(i % _DEPTH)


def kernel(x_nchw, w1, w2):
    B, C, H, W = x_nchw.shape
    hidden = w1.shape[0]
    HW = H * W
    dt = x_nchw.dtype

    x_flat = x_nchw.reshape(B, C, HW)
    w1t = w1.T.astype(jnp.float32)
    w2t = w2.T.astype(jnp.float32)

    ncores = 2 if B % 2 == 0 else 1
    bt = 2 if B % (2 * ncores) == 0 else 1
    steps = B // (bt * ncores)

    cost = pl.CostEstimate(
        flops=B * (4 * C * HW + 4 * C * hidden),
        transcendentals=B * C,
        bytes_accessed=2 * B * C * HW * jnp.dtype(dt).itemsize,
    )

    out_flat = pl.pallas_call(
        functools.partial(_se_ring_body, bt=bt, steps=steps, inv_hw=1.0 / HW),
        out_shape=jax.ShapeDtypeStruct((B, C, HW), dt),
        grid=(ncores,),
        in_specs=[
            pl.BlockSpec(memory_space=pl.ANY),
            pl.BlockSpec((C, hidden), lambda p: (0, 0)),
            pl.BlockSpec((hidden, C), lambda p: (0, 0)),
        ],
        out_specs=pl.BlockSpec(memory_space=pl.ANY),
        scratch_shapes=[
            pltpu.VMEM((_DEPTH, bt, C, HW), dt),
            pltpu.VMEM((_DEPTH, bt, C, HW), dt),
            pltpu.SemaphoreType.DMA((_DEPTH,)),
            pltpu.SemaphoreType.DMA((_DEPTH,)),
        ],
        compiler_params=pltpu.CompilerParams(
            dimension_semantics=("parallel",),
            vmem_limit_bytes=60 << 20,
        ),
        cost_estimate=cost,
    )(x_flat, w1t, w2t)

    return out_flat.reshape(B, C, H, W)


# speedup vs baseline: 1.4962x; 1.0120x over previous
import functools

import jax
import jax.numpy as jnp
from jax.experimental import pallas as pl
from jax.experimental.pallas import tpu as pltpu

_DEPTH = 4


def _se_ring_body(x_hbm, w1t_ref, w2t_ref, o_hbm, x_buf, o_buf,
                  in_sems, out_sems, *, bt, steps, inv_hw):
    base = pl.program_id(0) * steps

    def dma_in(slot, step):
        pltpu.make_async_copy(x_hbm.at[pl.ds((base + step) * bt, bt)],
                              x_buf.at[slot], in_sems.at[slot]).start()

    def wait_in(slot):
        pltpu.make_async_copy(x_buf.at[slot], x_buf.at[slot],
                              in_sems.at[slot]).wait()

    def dma_out(slot, step):
        pltpu.make_async_copy(o_buf.at[slot],
                              o_hbm.at[pl.ds((base + step) * bt, bt)],
                              out_sems.at[slot]).start(priority=1)

    def wait_out(slot):
        pltpu.make_async_copy(o_buf.at[slot], o_buf.at[slot],
                              out_sems.at[slot]).wait()

    for k in range(min(_DEPTH, steps)):
        dma_in(k, k)

    for i in range(steps):
        slot = i % _DEPTH
        wait_in(slot)
        if i >= _DEPTH:
            wait_out(slot)

        x = x_buf[slot]
        pooled = jnp.sum(x, axis=2, dtype=jnp.float32) * inv_hw
        hid = jnp.maximum(
            jnp.dot(pooled, w1t_ref[...],
                    preferred_element_type=jnp.float32), 0.0)
        gate = jax.nn.sigmoid(
            jnp.dot(hid, w2t_ref[...], preferred_element_type=jnp.float32))
        o_buf[slot] = x * gate.astype(x.dtype)[:, :, None]

        dma_out(slot, i)
        if i + _DEPTH < steps:
            dma_in(slot, i + _DEPTH)

    for i in range(max(steps - _DEPTH, 0), steps):
        wait_out(i % _DEPTH)


def kernel(x_nchw, w1, w2):
    B, C, H, W = x_nchw.shape
    hidden = w1.shape[0]
    HW = H * W
    dt = x_nchw.dtype

    x_flat = x_nchw.reshape(B, C, HW)
    w1t = w1.T.astype(jnp.float32)
    w2t = w2.T.astype(jnp.float32)

    ncores = 2 if B % 2 == 0 else 1
    bt = next(b for b in (4, 2, 1) if B % (b * ncores) == 0)
    steps = B // (bt * ncores)

    cost = pl.CostEstimate(
        flops=B * (4 * C * HW + 4 * C * hidden),
        transcendentals=B * C,
        bytes_accessed=2 * B * C * HW * jnp.dtype(dt).itemsize,
    )

    out_flat = pl.pallas_call(
        functools.partial(_se_ring_body, bt=bt, steps=steps, inv_hw=1.0 / HW),
        out_shape=jax.ShapeDtypeStruct((B, C, HW), dt),
        grid=(ncores,),
        in_specs=[
            pl.BlockSpec(memory_space=pl.ANY),
            pl.BlockSpec((C, hidden), lambda p: (0, 0)),
            pl.BlockSpec((hidden, C), lambda p: (0, 0)),
        ],
        out_specs=pl.BlockSpec(memory_space=pl.ANY),
        scratch_shapes=[
            pltpu.VMEM((_DEPTH, bt, C, HW), dt),
            pltpu.VMEM((_DEPTH, bt, C, HW), dt),
            pltpu.SemaphoreType.DMA((_DEPTH,)),
            pltpu.SemaphoreType.DMA((_DEPTH,)),
        ],
        compiler_params=pltpu.CompilerParams(
            dimension_semantics=("parallel",),
            vmem_limit_bytes=60 << 20,
        ),
        cost_estimate=cost,
    )(x_flat, w1t, w2t)

    return out_flat.reshape(B, C, H, W)
